# baseline (speedup 1.0000x reference)
import functools

import jax
import jax.numpy as jnp
from jax import lax
from jax.experimental import pallas as pl
from jax.experimental.pallas import tpu as pltpu
from jax.experimental.pallas import tpu_sc as plsc

N = 10000
E = 160000
ETOT = E + N
SEQ = 24
TRG = 12
F = 16
H0 = 4
NP = 10240
NC = 2
NT = 16
CH = 128
EPT = 10752
NCH = EPT // CH
EPAD = NT * EPT
RPT = NP // NT

_mesh = plsc.VectorSubcoreMesh(core_axis_name="c", subcore_axis_name="s",
                               num_cores=NC, num_subcores=NT)


def _edge_pass(width, heads):
    W = width
    WA = width + 16

    @functools.partial(
        pl.kernel,
        out_type=jax.ShapeDtypeStruct((NC, NP, WA), jnp.float32),
        mesh=_mesh,
        compiler_params=pltpu.CompilerParams(
            use_tc_tiling_on_sc=False, needs_layout_passes=False),
        scratch_types=[
            pltpu.VMEM((CH,), jnp.int32),
            pltpu.VMEM((CH,), jnp.int32),
            pltpu.VMEM((CH,), jnp.int32),
            pltpu.VMEM((CH, W), jnp.float32),
            pltpu.VMEM((CH, W), jnp.float32),
            pltpu.VMEM((CH, WA), jnp.float32),
            pltpu.VMEM((64, WA), jnp.float32),
            pltpu.VMEM((heads, 16), jnp.float32),
            pltpu.VMEM_SHARED((NP, WA), jnp.float32),
            pltpu.SemaphoreType.DMA,
        ],
    )
    def edge_kernel(src_hbm, dst_hbm, fs_hbm, fd_hbm, attn_hbm, out_hbm,
                    srcv, dstv, gidx, fsr, fdr, rows, zbuf, attn_v, acc, sem):
        c = lax.axis_index("c")
        s = lax.axis_index("s")
        pltpu.sync_copy(attn_hbm, attn_v)

        zv = jnp.zeros((16,), jnp.float32)

        def zero_stage(i, carry):
            for j in range(WA // 16):
                zbuf[i, pl.ds(j * 16, 16)] = zv
            return carry

        lax.fori_loop(0, 64, zero_stage, 0)

        def zero_acc(i, carry):
            pltpu.sync_copy(zbuf, acc.at[pl.ds(s * RPT + i * 64, 64)])
            return carry

        lax.fori_loop(0, RPT // 64, zero_acc, 0)
        plsc.subcore_barrier()

        attn_h = [attn_v[h] for h in range(heads)]
        lane = lax.broadcasted_iota(jnp.int32, (16,), 0)
        masks = [(lane == h).astype(jnp.float32) for h in range(heads)]
        off = c * NP

        def chunk(k, carry):
            base = s * EPT + k * CH
            pltpu.sync_copy(src_hbm.at[pl.ds(base, CH)], srcv)
            pltpu.sync_copy(dst_hbm.at[pl.ds(base, CH)], dstv)
            for j in range(CH // 16):
                gidx[pl.ds(j * 16, 16)] = srcv[pl.ds(j * 16, 16)] + off
            pltpu.async_copy(fs_hbm.at[gidx], fsr, sem).wait()
            for j in range(CH // 16):
                gidx[pl.ds(j * 16, 16)] = dstv[pl.ds(j * 16, 16)] + off
            pltpu.async_copy(fd_hbm.at[gidx], fdr, sem).wait()

            @plsc.parallel_loop(0, CH, unroll=8)
            def edge(e):
                den = None
                for h in range(heads):
                    a = fsr[e, pl.ds(h * 16, 16)]
                    b = fdr[e, pl.ds(h * 16, 16)]
                    q = a + b
                    ql = jnp.maximum(q, 0.0) + 0.2 * jnp.minimum(q, 0.0)
                    lg = jnp.sum(ql * attn_h[h])
                    ex = jnp.exp(jnp.broadcast_to(lg, (16,)))
                    rows[e, pl.ds(h * 16, 16)] = a * ex
                    dh = ex * masks[h]
                    den = dh if den is None else den + dh
                rows[e, pl.ds(W, 16)] = den
            pltpu.sync_copy(rows, acc.at[dstv], add=True)
            return carry

        lax.fori_loop(0, NCH, chunk, 0)
        plsc.subcore_barrier()
        pltpu.sync_copy(acc.at[pl.ds(s * RPT, RPT)],
                        out_hbm.at[c, pl.ds(s * RPT, RPT)])

    return edge_kernel


_edge0 = _edge_pass(64, 4)
_edge1 = _edge_pass(16, 1)


BP = 2048
NB = NP // BP


def _prep_body(xsel_ref, w0s_ref, b0s_ref, w0d_ref, b0d_ref,
               x_ref, fs_ref, fd_ref):
    a = xsel_ref[...]
    lane = lax.broadcasted_iota(jnp.int32, a.shape, 2)
    x = jnp.where(lane == 0, a, jnp.log(a + 1.0) * (1.0 / jnp.log(10.0)))
    x_ref[...] = x
    x2 = x.reshape(2 * BP, 16)
    fs_ref[...] = (jnp.dot(x2, w0s_ref[...], preferred_element_type=jnp.float32)
                   + b0s_ref[...]).reshape(2, BP, 64)
    fd_ref[...] = (jnp.dot(x2, w0d_ref[...], preferred_element_type=jnp.float32)
                   + b0d_ref[...]).reshape(2, BP, 64)


def _mid_body(acc_ref, w1s_ref, b1s_ref, w1d_ref, b1d_ref, fs1_ref, fd1_ref):
    r = acc_ref[...]
    hs = []
    for h in range(H0):
        num = r[..., h * 16:(h + 1) * 16]
        den = r[..., 64 + h:65 + h]
        hs.append(num / (den + 1e-9))
    h0 = jnp.concatenate(hs, axis=-1).reshape(2 * BP, 64)
    fs1_ref[...] = (jnp.dot(h0, w1s_ref[...], preferred_element_type=jnp.float32)
                    + b1s_ref[...]).reshape(2, BP, 16)
    fd1_ref[...] = (jnp.dot(h0, w1d_ref[...], preferred_element_type=jnp.float32)
                    + b1d_ref[...]).reshape(2, BP, 16)


def _post_body(acc1_ref, x_ref, src2_ref, ln0g_ref, ln0b_ref, encw_ref,
               dp_ref, out_ref):
    r = acc1_ref[...]
    h1 = r[..., :16] / (r[..., 16:17] + 1e-9)
    g = x_ref[...] + h1
    mu = jnp.mean(g, axis=-1, keepdims=True)
    var = jnp.mean((g - mu) ** 2, axis=-1, keepdims=True)
    ln = (g - mu) / jnp.sqrt(var + 1e-5) * ln0g_ref[...] + ln0b_ref[...]
    encs = jnp.sum(ln * encw_ref[...], axis=-1) + dp_ref[43]

    out_ref[pl.ds(0, 1), :] = encs[0:1, :]

    def lin2(a, b, i):
        u0 = a * dp_ref[i] + b * dp_ref[i + 2] + dp_ref[i + 4]
        u1 = a * dp_ref[i + 1] + b * dp_ref[i + 3] + dp_ref[i + 5]
        return u0, u1

    def minlstm(a, b, base):
        uf0, uf1 = lin2(a, b, base)
        ui0, ui1 = lin2(a, b, base + 6)
        ug0, ug1 = lin2(a, b, base + 12)
        f0 = jax.nn.sigmoid(uf0)
        f1 = jax.nn.sigmoid(uf1)
        i0 = jax.nn.sigmoid(ui0)
        i1 = jax.nn.sigmoid(ui1)
        return i0 * ug0 / (f0 + i0 + 1e-9), i1 * ug1 / (f1 + i1 + 1e-9)

    a = encs[1:2, :]
    for t in range(TRG):
        b = jnp.log(src2_ref[pl.ds(t, 1), :] + 1.0)
        h0, h1 = minlstm(a, b, 0)
        k0, k1 = minlstm(h0, h1, 18)
        r0 = a + k0
        r1 = b + k1
        d = (r0 - r1) * 0.5
        inv = 1.0 / jnp.sqrt(d * d + 1e-5)
        l0 = d * inv * dp_ref[36] + dp_ref[38]
        l1 = -d * inv * dp_ref[37] + dp_ref[39]
        o = l0 * dp_ref[40] + l1 * dp_ref[41] + dp_ref[42]
        out_ref[pl.ds(t + 1, 1), :] = o
        a = o


_f32 = jnp.float32


def _sds(shape):
    return jax.ShapeDtypeStruct(shape, _f32)


def _full(shape):
    return pl.BlockSpec(shape, lambda i: tuple(0 for _ in shape))


_prep = pl.pallas_call(
    _prep_body,
    grid=(NB,),
    in_specs=[
        pl.BlockSpec((2, BP, 16), lambda i: (0, i, 0)),
        _full((16, 64)), _full((64,)), _full((16, 64)), _full((64,)),
    ],
    out_specs=[
        pl.BlockSpec((2, BP, 16), lambda i: (0, i, 0)),
        pl.BlockSpec((2, BP, 64), lambda i: (0, i, 0)),
        pl.BlockSpec((2, BP, 64), lambda i: (0, i, 0)),
    ],
    out_shape=[_sds((2, NP, 16)), _sds((2, NP, 64)), _sds((2, NP, 64))],
)

_mid = pl.pallas_call(
    _mid_body,
    grid=(NB,),
    in_specs=[
        pl.BlockSpec((2, BP, 80), lambda i: (0, i, 0)),
        _full((64, 16)), _full((16,)), _full((64, 16)), _full((16,)),
    ],
    out_specs=[
        pl.BlockSpec((2, BP, 16), lambda i: (0, i, 0)),
        pl.BlockSpec((2, BP, 16), lambda i: (0, i, 0)),
    ],
    out_shape=[_sds((2, NP, 16)), _sds((2, NP, 16))],
)

_post = pl.pallas_call(
    _post_body,
    grid=(NB,),
    in_specs=[
        pl.BlockSpec((2, BP, 32), lambda i: (0, i, 0)),
        pl.BlockSpec((2, BP, 16), lambda i: (0, i, 0)),
        pl.BlockSpec((TRG, BP), lambda i: (0, i)),
        _full((16,)), _full((16,)), _full((16,)),
        pl.BlockSpec(memory_space=pltpu.SMEM),
    ],
    out_specs=pl.BlockSpec((TRG + 1, BP), lambda i: (0, i)),
    out_shape=_sds((TRG + 1, NP)),
)


def kernel(src1, src2, edge_index, params):
    p = params
    idt = edge_index.dtype

    xsel = jnp.transpose(src1[:, jnp.array([0, SEQ - 1]), :], (1, 0, 2))
    xsel = jnp.pad(xsel, ((0, 0), (0, NP - N), (0, 0)))
    loop = jnp.arange(N, dtype=idt)
    epad = jnp.full((EPAD - ETOT,), N, idt)
    srcf = jnp.concatenate([edge_index[0], loop, epad])
    dstf = jnp.concatenate([edge_index[1], loop, epad])
    src2t = jnp.pad(jnp.transpose(src2[:, :, 0]), ((0, 0), (0, NP - N)))

    dp = jnp.concatenate([
        p['lstm_Wf'].reshape(-1), p['lstm_bf'],
        p['lstm_Wi'].reshape(-1), p['lstm_bi'],
        p['lstm_Wh'].reshape(-1), p['lstm_bh'],
        p['lstm1_Wf'].reshape(-1), p['lstm1_bf'],
        p['lstm1_Wi'].reshape(-1), p['lstm1_bi'],
        p['lstm1_Wh'].reshape(-1), p['lstm1_bh'],
        p['ln1_g'], p['ln1_b'],
        p['fc_out_W'].reshape(-1), p['fc_out_b'],
        p['enc_fc_b'], jnp.zeros((4,), _f32),
    ])

    x_tbl, fs0, fd0 = _prep(xsel, p['gnn0_Wsrc'], p['gnn0_bsrc'],
                            p['gnn0_Wdst'], p['gnn0_bdst'])
    acc0 = _edge0(srcf, dstf, fs0.reshape(2 * NP, 64),
                  fd0.reshape(2 * NP, 64), p['gnn0_attn'])
    fs1, fd1 = _mid(acc0, p['gnn1_Wsrc'], p['gnn1_bsrc'],
                    p['gnn1_Wdst'], p['gnn1_bdst'])
    acc1 = _edge1(srcf, dstf, fs1.reshape(2 * NP, 16),
                  fd1.reshape(2 * NP, 16), p['gnn1_attn'])
    out13 = _post(acc1, x_tbl, src2t, p['ln0_g'], p['ln0_b'],
                  p['enc_fc_W'][:, 0], dp)
    return jnp.transpose(out13[:, :N])[:, :, None]


# speedup vs baseline: 1.0012x; 1.0012x over previous
import functools

import jax
import jax.numpy as jnp
from jax import lax
from jax.experimental import pallas as pl
from jax.experimental.pallas import tpu as pltpu
from jax.experimental.pallas import tpu_sc as plsc

N = 10000
E = 160000
ETOT = E + N
SEQ = 24
TRG = 12
F = 16
H0 = 4
NP = 10240
NC = 2
NT = 16
CH = 128
EPT = 10752
NCH = EPT // CH
EPAD = NT * EPT
RPT = NP // NT

_mesh = plsc.VectorSubcoreMesh(core_axis_name="c", subcore_axis_name="s",
                               num_cores=NC, num_subcores=NT)


def _edge_pass(width, heads):
    W = width
    WA = width + 16

    @functools.partial(
        pl.kernel,
        out_type=jax.ShapeDtypeStruct((NC, NP, WA), jnp.float32),
        mesh=_mesh,
        compiler_params=pltpu.CompilerParams(
            use_tc_tiling_on_sc=False, needs_layout_passes=False),
        scratch_types=[
            pltpu.VMEM((CH,), jnp.int32),
            pltpu.VMEM((CH,), jnp.int32),
            pltpu.VMEM((CH,), jnp.int32),
            pltpu.VMEM((CH, W), jnp.float32),
            pltpu.VMEM((CH, W), jnp.float32),
            pltpu.VMEM((CH, WA), jnp.float32),
            pltpu.VMEM((64, WA), jnp.float32),
            pltpu.VMEM((heads, 16), jnp.float32),
            pltpu.VMEM_SHARED((NP, WA), jnp.float32),
            pltpu.SemaphoreType.DMA,
        ],
    )
    def edge_kernel(src_hbm, dst_hbm, fs_hbm, fd_hbm, attn_hbm, out_hbm,
                    srcv, dstv, gidx, fsr, fdr, rows, zbuf, attn_v, acc, sem):
        c = lax.axis_index("c")
        s = lax.axis_index("s")
        pltpu.sync_copy(attn_hbm, attn_v)

        zv = jnp.zeros((16,), jnp.float32)

        def zero_stage(i, carry):
            for j in range(WA // 16):
                zbuf[i, pl.ds(j * 16, 16)] = zv
            return carry

        lax.fori_loop(0, 64, zero_stage, 0)

        def zero_acc(i, carry):
            pltpu.sync_copy(zbuf, acc.at[pl.ds(s * RPT + i * 64, 64)])
            return carry

        lax.fori_loop(0, RPT // 64, zero_acc, 0)
        plsc.subcore_barrier()

        attn_h = [attn_v[h] for h in range(heads)]
        lane = lax.broadcasted_iota(jnp.int32, (16,), 0)
        masks = [(lane == h).astype(jnp.float32) for h in range(heads)]
        off = c * NP

        def chunk(k, carry):
            base = s * EPT + k * CH
            pltpu.sync_copy(src_hbm.at[pl.ds(base, CH)], srcv)
            pltpu.sync_copy(dst_hbm.at[pl.ds(base, CH)], dstv)
            for j in range(CH // 16):
                gidx[pl.ds(j * 16, 16)] = srcv[pl.ds(j * 16, 16)] + off
            pltpu.async_copy(fs_hbm.at[gidx], fsr, sem).wait()
            for j in range(CH // 16):
                gidx[pl.ds(j * 16, 16)] = dstv[pl.ds(j * 16, 16)] + off
            pltpu.async_copy(fd_hbm.at[gidx], fdr, sem).wait()

            @plsc.parallel_loop(0, CH, unroll=4)
            def edge(e):
                den = None
                for h in range(heads):
                    a = fsr[e, pl.ds(h * 16, 16)]
                    b = fdr[e, pl.ds(h * 16, 16)]
                    q = a + b
                    ql = jnp.maximum(q, 0.0) + 0.2 * jnp.minimum(q, 0.0)
                    lg = jnp.sum(ql * attn_h[h])
                    ex = jnp.exp(jnp.broadcast_to(lg, (16,)))
                    rows[e, pl.ds(h * 16, 16)] = a * ex
                    dh = ex * masks[h]
                    den = dh if den is None else den + dh
                rows[e, pl.ds(W, 16)] = den
            pltpu.sync_copy(rows, acc.at[dstv], add=True)
            return carry

        lax.fori_loop(0, NCH, chunk, 0)
        plsc.subcore_barrier()
        pltpu.sync_copy(acc.at[pl.ds(s * RPT, RPT)],
                        out_hbm.at[c, pl.ds(s * RPT, RPT)])

    return edge_kernel


_edge0 = _edge_pass(64, 4)
_edge1 = _edge_pass(16, 1)


BP = 2048
NB = NP // BP


def _prep_body(xsel_ref, w0s_ref, b0s_ref, w0d_ref, b0d_ref,
               x_ref, fs_ref, fd_ref):
    a = xsel_ref[...]
    lane = lax.broadcasted_iota(jnp.int32, a.shape, 2)
    x = jnp.where(lane == 0, a, jnp.log(a + 1.0) * (1.0 / jnp.log(10.0)))
    x_ref[...] = x
    x2 = x.reshape(2 * BP, 16)
    fs_ref[...] = (jnp.dot(x2, w0s_ref[...], preferred_element_type=jnp.float32)
                   + b0s_ref[...]).reshape(2, BP, 64)
    fd_ref[...] = (jnp.dot(x2, w0d_ref[...], preferred_element_type=jnp.float32)
                   + b0d_ref[...]).reshape(2, BP, 64)


def _mid_body(acc_ref, w1s_ref, b1s_ref, w1d_ref, b1d_ref, fs1_ref, fd1_ref):
    r = acc_ref[...]
    hs = []
    for h in range(H0):
        num = r[..., h * 16:(h + 1) * 16]
        den = r[..., 64 + h:65 + h]
        hs.append(num / (den + 1e-9))
    h0 = jnp.concatenate(hs, axis=-1).reshape(2 * BP, 64)
    fs1_ref[...] = (jnp.dot(h0, w1s_ref[...], preferred_element_type=jnp.float32)
                    + b1s_ref[...]).reshape(2, BP, 16)
    fd1_ref[...] = (jnp.dot(h0, w1d_ref[...], preferred_element_type=jnp.float32)
                    + b1d_ref[...]).reshape(2, BP, 16)


def _post_body(acc1_ref, x_ref, src2_ref, ln0g_ref, ln0b_ref, encw_ref,
               dp_ref, out_ref):
    r = acc1_ref[...]
    h1 = r[..., :16] / (r[..., 16:17] + 1e-9)
    g = x_ref[...] + h1
    mu = jnp.mean(g, axis=-1, keepdims=True)
    var = jnp.mean((g - mu) ** 2, axis=-1, keepdims=True)
    ln = (g - mu) / jnp.sqrt(var + 1e-5) * ln0g_ref[...] + ln0b_ref[...]
    encs = jnp.sum(ln * encw_ref[...], axis=-1) + dp_ref[43]

    out_ref[pl.ds(0, 1), :] = encs[0:1, :]

    def lin2(a, b, i):
        u0 = a * dp_ref[i] + b * dp_ref[i + 2] + dp_ref[i + 4]
        u1 = a * dp_ref[i + 1] + b * dp_ref[i + 3] + dp_ref[i + 5]
        return u0, u1

    def minlstm(a, b, base):
        uf0, uf1 = lin2(a, b, base)
        ui0, ui1 = lin2(a, b, base + 6)
        ug0, ug1 = lin2(a, b, base + 12)
        f0 = jax.nn.sigmoid(uf0)
        f1 = jax.nn.sigmoid(uf1)
        i0 = jax.nn.sigmoid(ui0)
        i1 = jax.nn.sigmoid(ui1)
        return i0 * ug0 / (f0 + i0 + 1e-9), i1 * ug1 / (f1 + i1 + 1e-9)

    a = encs[1:2, :]
    for t in range(TRG):
        b = jnp.log(src2_ref[pl.ds(t, 1), :] + 1.0)
        h0, h1 = minlstm(a, b, 0)
        k0, k1 = minlstm(h0, h1, 18)
        r0 = a + k0
        r1 = b + k1
        d = (r0 - r1) * 0.5
        inv = 1.0 / jnp.sqrt(d * d + 1e-5)
        l0 = d * inv * dp_ref[36] + dp_ref[38]
        l1 = -d * inv * dp_ref[37] + dp_ref[39]
        o = l0 * dp_ref[40] + l1 * dp_ref[41] + dp_ref[42]
        out_ref[pl.ds(t + 1, 1), :] = o
        a = o


_f32 = jnp.float32


def _sds(shape):
    return jax.ShapeDtypeStruct(shape, _f32)


def _full(shape):
    return pl.BlockSpec(shape, lambda i: tuple(0 for _ in shape))


_prep = pl.pallas_call(
    _prep_body,
    grid=(NB,),
    in_specs=[
        pl.BlockSpec((2, BP, 16), lambda i: (0, i, 0)),
        _full((16, 64)), _full((64,)), _full((16, 64)), _full((64,)),
    ],
    out_specs=[
        pl.BlockSpec((2, BP, 16), lambda i: (0, i, 0)),
        pl.BlockSpec((2, BP, 64), lambda i: (0, i, 0)),
        pl.BlockSpec((2, BP, 64), lambda i: (0, i, 0)),
    ],
    out_shape=[_sds((2, NP, 16)), _sds((2, NP, 64)), _sds((2, NP, 64))],
)

_mid = pl.pallas_call(
    _mid_body,
    grid=(NB,),
    in_specs=[
        pl.BlockSpec((2, BP, 80), lambda i: (0, i, 0)),
        _full((64, 16)), _full((16,)), _full((64, 16)), _full((16,)),
    ],
    out_specs=[
        pl.BlockSpec((2, BP, 16), lambda i: (0, i, 0)),
        pl.BlockSpec((2, BP, 16), lambda i: (0, i, 0)),
    ],
    out_shape=[_sds((2, NP, 16)), _sds((2, NP, 16))],
)

_post = pl.pallas_call(
    _post_body,
    grid=(NB,),
    in_specs=[
        pl.BlockSpec((2, BP, 32), lambda i: (0, i, 0)),
        pl.BlockSpec((2, BP, 16), lambda i: (0, i, 0)),
        pl.BlockSpec((TRG, BP), lambda i: (0, i)),
        _full((16,)), _full((16,)), _full((16,)),
        pl.BlockSpec(memory_space=pltpu.SMEM),
    ],
    out_specs=pl.BlockSpec((TRG + 1, BP), lambda i: (0, i)),
    out_shape=_sds((TRG + 1, NP)),
)


def kernel(src1, src2, edge_index, params):
    p = params
    idt = edge_index.dtype

    xsel = jnp.transpose(src1[:, jnp.array([0, SEQ - 1]), :], (1, 0, 2))
    xsel = jnp.pad(xsel, ((0, 0), (0, NP - N), (0, 0)))
    loop = jnp.arange(N, dtype=idt)
    epad = jnp.full((EPAD - ETOT,), N, idt)
    srcf = jnp.concatenate([edge_index[0], loop, epad])
    dstf = jnp.concatenate([edge_index[1], loop, epad])
    src2t = jnp.pad(jnp.transpose(src2[:, :, 0]), ((0, 0), (0, NP - N)))

    dp = jnp.concatenate([
        p['lstm_Wf'].reshape(-1), p['lstm_bf'],
        p['lstm_Wi'].reshape(-1), p['lstm_bi'],
        p['lstm_Wh'].reshape(-1), p['lstm_bh'],
        p['lstm1_Wf'].reshape(-1), p['lstm1_bf'],
        p['lstm1_Wi'].reshape(-1), p['lstm1_bi'],
        p['lstm1_Wh'].reshape(-1), p['lstm1_bh'],
        p['ln1_g'], p['ln1_b'],
        p['fc_out_W'].reshape(-1), p['fc_out_b'],
        p['enc_fc_b'], jnp.zeros((4,), _f32),
    ])

    x_tbl, fs0, fd0 = _prep(xsel, p['gnn0_Wsrc'], p['gnn0_bsrc'],
                            p['gnn0_Wdst'], p['gnn0_bdst'])
    acc0 = _edge0(srcf, dstf, fs0.reshape(2 * NP, 64),
                  fd0.reshape(2 * NP, 64), p['gnn0_attn'])
    fs1, fd1 = _mid(acc0, p['gnn1_Wsrc'], p['gnn1_bsrc'],
                    p['gnn1_Wdst'], p['gnn1_bdst'])
    acc1 = _edge1(srcf, dstf, fs1.reshape(2 * NP, 16),
                  fd1.reshape(2 * NP, 16), p['gnn1_attn'])
    out13 = _post(acc1, x_tbl, src2t, p['ln0_g'], p['ln0_b'],
                  p['enc_fc_W'][:, 0], dp)
    return jnp.transpose(out13[:, :N])[:, :, None]


# speedup vs baseline: 1.4648x; 1.4630x over previous
import functools

import jax
import jax.numpy as jnp
from jax import lax
from jax.experimental import pallas as pl
from jax.experimental.pallas import tpu as pltpu
from jax.experimental.pallas import tpu_sc as plsc

N = 10000
E = 160000
ETOT = E + N
SEQ = 24
TRG = 12
F = 16
H0 = 4
NP = 10240
NC = 2
NT = 16
CH = 128
EPT = 10752
NCH = EPT // CH
EPAD = NT * EPT
RPT = NP // NT

_mesh = plsc.VectorSubcoreMesh(core_axis_name="c", subcore_axis_name="s",
                               num_cores=NC, num_subcores=NT)


def _edge_pass(width, heads):
    W = width
    WA = width + 16

    @functools.partial(
        pl.kernel,
        out_type=jax.ShapeDtypeStruct((NC, NP, WA), jnp.float32),
        mesh=_mesh,
        compiler_params=pltpu.CompilerParams(
            use_tc_tiling_on_sc=False, needs_layout_passes=False),
        scratch_types=[
            pltpu.VMEM((CH,), jnp.int32),
            pltpu.VMEM((CH,), jnp.int32),
            pltpu.VMEM((CH,), jnp.int32),
            pltpu.VMEM((CH, W), jnp.float32),
            pltpu.VMEM((CH, W), jnp.float32),
            pltpu.VMEM((CH, WA), jnp.float32),
            pltpu.VMEM((64, WA), jnp.float32),
            pltpu.VMEM((heads, 16), jnp.float32),
            pltpu.VMEM_SHARED((NP, WA), jnp.float32),
            pltpu.SemaphoreType.DMA,
        ],
    )
    def edge_kernel(src_hbm, dst_hbm, fs_hbm, fd_hbm, attn_hbm, out_hbm,
                    srcv, dstv, gidx, fsr, fdr, rows, zbuf, attn_v, acc, sem):
        c = lax.axis_index("c")
        s = lax.axis_index("s")
        pltpu.sync_copy(attn_hbm, attn_v)

        zv = jnp.zeros((16,), jnp.float32)

        def zero_stage(i, carry):
            for j in range(WA // 16):
                zbuf[i, pl.ds(j * 16, 16)] = zv
            return carry

        lax.fori_loop(0, 64, zero_stage, 0)

        def zero_acc(i, carry):
            pltpu.sync_copy(zbuf, acc.at[pl.ds(s * RPT + i * 64, 64)])
            return carry

        lax.fori_loop(0, RPT // 64, zero_acc, 0)
        plsc.subcore_barrier()

        attn_h = [attn_v[h] for h in range(heads)]
        lane = lax.broadcasted_iota(jnp.int32, (16,), 0)
        masks = [(lane == h).astype(jnp.float32) for h in range(heads)]
        off = c * NP

        def chunk(k, carry):
            base = s * EPT + k * CH
            pltpu.sync_copy(src_hbm.at[pl.ds(base, CH)], srcv)
            pltpu.sync_copy(dst_hbm.at[pl.ds(base, CH)], dstv)
            for j in range(CH // 16):
                gidx[pl.ds(j * 16, 16)] = srcv[pl.ds(j * 16, 16)] + off
            pltpu.async_copy(fs_hbm.at[gidx], fsr, sem).wait()
            for j in range(CH // 16):
                gidx[pl.ds(j * 16, 16)] = dstv[pl.ds(j * 16, 16)] + off
            pltpu.async_copy(fd_hbm.at[gidx], fdr, sem).wait()

            @plsc.parallel_loop(0, CH, unroll=4)
            def edge(e):
                den = None
                for h in range(heads):
                    a = fsr[e, pl.ds(h * 16, 16)]
                    b = fdr[e, pl.ds(h * 16, 16)]
                    q = a + b
                    ql = jnp.maximum(q, 0.0) + 0.2 * jnp.minimum(q, 0.0)
                    lg = jnp.sum(ql * attn_h[h])
                    ex = jnp.exp(jnp.broadcast_to(lg, (16,)))
                    rows[e, pl.ds(h * 16, 16)] = a * ex
                    dh = ex * masks[h]
                    den = dh if den is None else den + dh
                rows[e, pl.ds(W, 16)] = den
            pltpu.sync_copy(rows, acc.at[dstv], add=True)
            return carry

        lax.fori_loop(0, NCH, chunk, 0)
        plsc.subcore_barrier()
        pltpu.sync_copy(acc.at[pl.ds(s * RPT, RPT)],
                        out_hbm.at[c, pl.ds(s * RPT, RPT)])

    return edge_kernel


_edge0 = _edge_pass(64, 4)
_edge1 = _edge_pass(16, 1)


BP = 2048
NB = NP // BP


def _prep_body(xsel_ref, w0s_ref, b0s_ref, w0d_ref, b0d_ref,
               x_ref, fs_ref, fd_ref):
    a = xsel_ref[...]
    lane = lax.broadcasted_iota(jnp.int32, a.shape, 2)
    x = jnp.where(lane == 0, a, jnp.log(a + 1.0) * (1.0 / jnp.log(10.0)))
    x_ref[...] = x
    x2 = x.reshape(2 * BP, 16)
    fs_ref[...] = (jnp.dot(x2, w0s_ref[...], preferred_element_type=jnp.float32)
                   + b0s_ref[...]).reshape(2, BP, 64)
    fd_ref[...] = (jnp.dot(x2, w0d_ref[...], preferred_element_type=jnp.float32)
                   + b0d_ref[...]).reshape(2, BP, 64)


def _mid_body(acc_ref, w1s_ref, b1s_ref, w1d_ref, b1d_ref, fs1_ref, fd1_ref):
    r = acc_ref[...]
    hs = []
    for h in range(H0):
        num = r[..., h * 16:(h + 1) * 16]
        den = r[..., 64 + h:65 + h]
        hs.append(num / (den + 1e-9))
    h0 = jnp.concatenate(hs, axis=-1).reshape(2 * BP, 64)
    fs1_ref[...] = (jnp.dot(h0, w1s_ref[...], preferred_element_type=jnp.float32)
                    + b1s_ref[...]).reshape(2, BP, 16)
    fd1_ref[...] = (jnp.dot(h0, w1d_ref[...], preferred_element_type=jnp.float32)
                    + b1d_ref[...]).reshape(2, BP, 16)


def _post_body(acc1_ref, x_ref, ln0g_ref, ln0b_ref, encw_ref, dp_ref,
               enc_ref):
    r = acc1_ref[...]
    h1 = r[..., :16] / (r[..., 16:17] + 1e-9)
    g = x_ref[...] + h1
    mu = jnp.mean(g, axis=-1, keepdims=True)
    var = jnp.mean((g - mu) ** 2, axis=-1, keepdims=True)
    ln = (g - mu) / jnp.sqrt(var + 1e-5) * ln0g_ref[...] + ln0b_ref[...]
    enc_ref[...] = jnp.sum(ln * encw_ref[...], axis=-1) + dp_ref[43]


NR = 8
NL = NP // NR


def _dec_body(enc_ref, src2_ref, dp_ref, out_ref):
    out_ref[pl.ds(0, 1)] = enc_ref[pl.ds(0, 1)]

    def lin2(a, b, i):
        u0 = a * dp_ref[i] + b * dp_ref[i + 2] + dp_ref[i + 4]
        u1 = a * dp_ref[i + 1] + b * dp_ref[i + 3] + dp_ref[i + 5]
        return u0, u1

    def minlstm(a, b, base):
        uf0, uf1 = lin2(a, b, base)
        ui0, ui1 = lin2(a, b, base + 6)
        ug0, ug1 = lin2(a, b, base + 12)
        f0 = jax.nn.sigmoid(uf0)
        f1 = jax.nn.sigmoid(uf1)
        i0 = jax.nn.sigmoid(ui0)
        i1 = jax.nn.sigmoid(ui1)
        return i0 * ug0 / (f0 + i0 + 1e-9), i1 * ug1 / (f1 + i1 + 1e-9)

    a = enc_ref[1]
    for t in range(TRG):
        b = jnp.log(src2_ref[t] + 1.0)
        h0, h1 = minlstm(a, b, 0)
        k0, k1 = minlstm(h0, h1, 18)
        r0 = a + k0
        r1 = b + k1
        d = (r0 - r1) * 0.5
        inv = 1.0 / jnp.sqrt(d * d + 1e-5)
        l0 = d * inv * dp_ref[36] + dp_ref[38]
        l1 = -d * inv * dp_ref[37] + dp_ref[39]
        o = l0 * dp_ref[40] + l1 * dp_ref[41] + dp_ref[42]
        out_ref[pl.ds(t + 1, 1)] = o[None]
        a = o


_f32 = jnp.float32


def _sds(shape):
    return jax.ShapeDtypeStruct(shape, _f32)


def _full(shape):
    return pl.BlockSpec(shape, lambda i: tuple(0 for _ in shape))


_prep = pl.pallas_call(
    _prep_body,
    grid=(NB,),
    in_specs=[
        pl.BlockSpec((2, BP, 16), lambda i: (0, i, 0)),
        _full((16, 64)), _full((64,)), _full((16, 64)), _full((64,)),
    ],
    out_specs=[
        pl.BlockSpec((2, BP, 16), lambda i: (0, i, 0)),
        pl.BlockSpec((2, BP, 64), lambda i: (0, i, 0)),
        pl.BlockSpec((2, BP, 64), lambda i: (0, i, 0)),
    ],
    out_shape=[_sds((2, NP, 16)), _sds((2, NP, 64)), _sds((2, NP, 64))],
)

_mid = pl.pallas_call(
    _mid_body,
    grid=(NB,),
    in_specs=[
        pl.BlockSpec((2, BP, 80), lambda i: (0, i, 0)),
        _full((64, 16)), _full((16,)), _full((64, 16)), _full((16,)),
    ],
    out_specs=[
        pl.BlockSpec((2, BP, 16), lambda i: (0, i, 0)),
        pl.BlockSpec((2, BP, 16), lambda i: (0, i, 0)),
    ],
    out_shape=[_sds((2, NP, 16)), _sds((2, NP, 16))],
)

_post = pl.pallas_call(
    _post_body,
    grid=(NB,),
    in_specs=[
        pl.BlockSpec((2, BP, 32), lambda i: (0, i, 0)),
        pl.BlockSpec((2, BP, 16), lambda i: (0, i, 0)),
        _full((16,)), _full((16,)), _full((16,)),
        pl.BlockSpec(memory_space=pltpu.SMEM),
    ],
    out_specs=pl.BlockSpec((2, BP), lambda i: (0, i)),
    out_shape=_sds((2, NP)),
)

_dec = pl.pallas_call(
    _dec_body,
    in_specs=[
        pl.BlockSpec((2, NR, NL), lambda: (0, 0, 0)),
        pl.BlockSpec((TRG, NR, NL), lambda: (0, 0, 0)),
        pl.BlockSpec(memory_space=pltpu.SMEM),
    ],
    out_specs=pl.BlockSpec((TRG + 1, NR, NL), lambda: (0, 0, 0)),
    out_shape=_sds((TRG + 1, NR, NL)),
)


def kernel(src1, src2, edge_index, params):
    p = params
    idt = edge_index.dtype

    xsel = jnp.transpose(src1[:, jnp.array([0, SEQ - 1]), :], (1, 0, 2))
    xsel = jnp.pad(xsel, ((0, 0), (0, NP - N), (0, 0)))
    loop = jnp.arange(N, dtype=idt)
    epad = jnp.full((EPAD - ETOT,), N, idt)
    srcf = jnp.concatenate([edge_index[0], loop, epad])
    dstf = jnp.concatenate([edge_index[1], loop, epad])
    src2t = jnp.pad(jnp.transpose(src2[:, :, 0]), ((0, 0), (0, NP - N)))

    dp = jnp.concatenate([
        p['lstm_Wf'].reshape(-1), p['lstm_bf'],
        p['lstm_Wi'].reshape(-1), p['lstm_bi'],
        p['lstm_Wh'].reshape(-1), p['lstm_bh'],
        p['lstm1_Wf'].reshape(-1), p['lstm1_bf'],
        p['lstm1_Wi'].reshape(-1), p['lstm1_bi'],
        p['lstm1_Wh'].reshape(-1), p['lstm1_bh'],
        p['ln1_g'], p['ln1_b'],
        p['fc_out_W'].reshape(-1), p['fc_out_b'],
        p['enc_fc_b'], jnp.zeros((4,), _f32),
    ])

    x_tbl, fs0, fd0 = _prep(xsel, p['gnn0_Wsrc'], p['gnn0_bsrc'],
                            p['gnn0_Wdst'], p['gnn0_bdst'])
    acc0 = _edge0(srcf, dstf, fs0.reshape(2 * NP, 64),
                  fd0.reshape(2 * NP, 64), p['gnn0_attn'])
    fs1, fd1 = _mid(acc0, p['gnn1_Wsrc'], p['gnn1_bsrc'],
                    p['gnn1_Wdst'], p['gnn1_bdst'])
    acc1 = _edge1(srcf, dstf, fs1.reshape(2 * NP, 16),
                  fd1.reshape(2 * NP, 16), p['gnn1_attn'])
    encs = _post(acc1, x_tbl, p['ln0_g'], p['ln0_b'], p['enc_fc_W'][:, 0], dp)
    out13 = _dec(encs.reshape(2, NR, NL), src2t.reshape(TRG, NR, NL), dp)
    return jnp.transpose(out13.reshape(TRG + 1, NP)[:, :N])[:, :, None]


# speedup vs baseline: 2.2896x; 1.5631x over previous
import functools

import jax
import jax.numpy as jnp
from jax import lax
from jax.experimental import pallas as pl
from jax.experimental.pallas import tpu as pltpu
from jax.experimental.pallas import tpu_sc as plsc

N = 10000
E = 160000
ETOT = E + N
SEQ = 24
TRG = 12
F = 16
H0 = 4
NP = 10240
NC = 2
NT = 16
CH = 128
EPT = 10752
NCH = EPT // CH
EPAD = NT * EPT
RPT = NP // NT

_mesh = plsc.VectorSubcoreMesh(core_axis_name="c", subcore_axis_name="s",
                               num_cores=NC, num_subcores=NT)


def _edge_pass(width, heads):
    W = width
    WA = width + 16

    @functools.partial(
        pl.kernel,
        out_type=jax.ShapeDtypeStruct((NC, NP, WA), jnp.float32),
        mesh=_mesh,
        compiler_params=pltpu.CompilerParams(
            use_tc_tiling_on_sc=False, needs_layout_passes=False),
        scratch_types=[
            pltpu.VMEM((EPT,), jnp.int32),
            pltpu.VMEM((EPT,), jnp.int32),
            pltpu.VMEM((CH,), jnp.int32),
            pltpu.VMEM((CH,), jnp.int32),
            pltpu.VMEM((CH, W), jnp.float32),
            pltpu.VMEM((CH, W), jnp.float32),
            pltpu.VMEM((CH, W), jnp.float32),
            pltpu.VMEM((CH, W), jnp.float32),
            pltpu.VMEM((CH, WA), jnp.float32),
            pltpu.VMEM((64, WA), jnp.float32),
            pltpu.VMEM((heads, 16), jnp.float32),
            pltpu.VMEM_SHARED((NP, WA), jnp.float32),
            pltpu.SemaphoreType.DMA,
            pltpu.SemaphoreType.DMA,
        ],
    )
    def edge_kernel(src_hbm, dst_hbm, fs_hbm, fd_hbm, attn_hbm, out_hbm,
                    srcall, dstall, gd0, gd1, fsr0, fdr0, fsr1, fdr1,
                    rows, zbuf, attn_v, acc, sem0, sem1):
        c = lax.axis_index("c")
        s = lax.axis_index("s")
        pltpu.sync_copy(attn_hbm, attn_v)

        zv = jnp.zeros((16,), jnp.float32)

        def zero_stage(i, carry):
            for j in range(WA // 16):
                zbuf[i, pl.ds(j * 16, 16)] = zv
            return carry

        lax.fori_loop(0, 64, zero_stage, 0)

        def zero_acc(i, carry):
            pltpu.sync_copy(zbuf, acc.at[pl.ds(s * RPT + i * 64, 64)])
            return carry

        lax.fori_loop(0, RPT // 64, zero_acc, 0)

        pltpu.sync_copy(src_hbm.at[pl.ds(s * EPT, EPT)], srcall)
        pltpu.sync_copy(dst_hbm.at[pl.ds(s * EPT, EPT)], dstall)
        off = c * NP

        @plsc.parallel_loop(0, EPT // 16, unroll=8)
        def addoff(i):
            srcall[pl.ds(i * 16, 16)] = srcall[pl.ds(i * 16, 16)] + off

        plsc.subcore_barrier()

        attn_h = [attn_v[h] for h in range(heads)]
        lane = lax.broadcasted_iota(jnp.int32, (16,), 0)
        masks = [(lane == h).astype(jnp.float32) for h in range(heads)]

        def issue(k, gd, fsr, fdr, sem):
            for j in range(CH // 16):
                gd[pl.ds(j * 16, 16)] = dstall[pl.ds(k * CH + j * 16, 16)] + off
            hs = pltpu.async_copy(fs_hbm.at[srcall.at[pl.ds(k * CH, CH)]],
                                  fsr, sem)
            hd = pltpu.async_copy(fd_hbm.at[gd], fdr, sem)
            return hs, hd

        def consume(k, fsr, fdr):
            @plsc.parallel_loop(0, CH, unroll=4)
            def edge(e):
                den = None
                for h in range(heads):
                    a = fsr[e, pl.ds(h * 16, 16)]
                    b = fdr[e, pl.ds(h * 16, 16)]
                    q = a + b
                    ql = jnp.maximum(q, 0.0) + 0.2 * jnp.minimum(q, 0.0)
                    lg = jnp.sum(ql * attn_h[h])
                    ex = jnp.exp(jnp.broadcast_to(lg, (16,)))
                    rows[e, pl.ds(h * 16, 16)] = a * ex
                    dh = ex * masks[h]
                    den = dh if den is None else den + dh
                rows[e, pl.ds(W, 16)] = den
            pltpu.sync_copy(rows, acc.at[dstall.at[pl.ds(k * CH, CH)]],
                            add=True)

        def chunk2(j, carry):
            a = 2 * j
            b = a + 1
            ha_s, ha_d = issue(a, gd0, fsr0, fdr0, sem0)
            hb_s, hb_d = issue(b, gd1, fsr1, fdr1, sem1)
            ha_s.wait()
            ha_d.wait()
            consume(a, fsr0, fdr0)
            hb_s.wait()
            hb_d.wait()
            consume(b, fsr1, fdr1)
            return carry

        lax.fori_loop(0, NCH // 2, chunk2, 0)
        plsc.subcore_barrier()
        pltpu.sync_copy(acc.at[pl.ds(s * RPT, RPT)],
                        out_hbm.at[c, pl.ds(s * RPT, RPT)])

    return edge_kernel


_edge0 = _edge_pass(64, 4)
_edge1 = _edge_pass(16, 1)


BP = 2048
NB = NP // BP


def _prep_body(xsel_ref, w0s_ref, b0s_ref, w0d_ref, b0d_ref,
               x_ref, fs_ref, fd_ref):
    a = xsel_ref[...]
    lane = lax.broadcasted_iota(jnp.int32, a.shape, 2)
    x = jnp.where(lane == 0, a, jnp.log(a + 1.0) * (1.0 / jnp.log(10.0)))
    x_ref[...] = x
    x2 = x.reshape(2 * BP, 16)
    fs_ref[...] = (jnp.dot(x2, w0s_ref[...], preferred_element_type=jnp.float32)
                   + b0s_ref[...]).reshape(2, BP, 64)
    fd_ref[...] = (jnp.dot(x2, w0d_ref[...], preferred_element_type=jnp.float32)
                   + b0d_ref[...]).reshape(2, BP, 64)


def _mid_body(acc_ref, w1s_ref, b1s_ref, w1d_ref, b1d_ref, fs1_ref, fd1_ref):
    r = acc_ref[...]
    hs = []
    for h in range(H0):
        num = r[..., h * 16:(h + 1) * 16]
        den = r[..., 64 + h:65 + h]
        hs.append(num / (den + 1e-9))
    h0 = jnp.concatenate(hs, axis=-1).reshape(2 * BP, 64)
    fs1_ref[...] = (jnp.dot(h0, w1s_ref[...], preferred_element_type=jnp.float32)
                    + b1s_ref[...]).reshape(2, BP, 16)
    fd1_ref[...] = (jnp.dot(h0, w1d_ref[...], preferred_element_type=jnp.float32)
                    + b1d_ref[...]).reshape(2, BP, 16)


def _post_body(acc1_ref, x_ref, ln0g_ref, ln0b_ref, encw_ref, dp_ref,
               enc_ref):
    r = acc1_ref[...]
    h1 = r[..., :16] / (r[..., 16:17] + 1e-9)
    g = x_ref[...] + h1
    mu = jnp.mean(g, axis=-1, keepdims=True)
    var = jnp.mean((g - mu) ** 2, axis=-1, keepdims=True)
    ln = (g - mu) / jnp.sqrt(var + 1e-5) * ln0g_ref[...] + ln0b_ref[...]
    enc_ref[...] = jnp.sum(ln * encw_ref[...], axis=-1) + dp_ref[43]


NR = 8
NL = NP // NR


def _dec_body(enc_ref, src2_ref, dp_ref, out_ref):
    out_ref[pl.ds(0, 1)] = enc_ref[pl.ds(0, 1)]

    def lin2(a, b, i):
        u0 = a * dp_ref[i] + b * dp_ref[i + 2] + dp_ref[i + 4]
        u1 = a * dp_ref[i + 1] + b * dp_ref[i + 3] + dp_ref[i + 5]
        return u0, u1

    def minlstm(a, b, base):
        uf0, uf1 = lin2(a, b, base)
        ui0, ui1 = lin2(a, b, base + 6)
        ug0, ug1 = lin2(a, b, base + 12)
        f0 = jax.nn.sigmoid(uf0)
        f1 = jax.nn.sigmoid(uf1)
        i0 = jax.nn.sigmoid(ui0)
        i1 = jax.nn.sigmoid(ui1)
        return i0 * ug0 / (f0 + i0 + 1e-9), i1 * ug1 / (f1 + i1 + 1e-9)

    a = enc_ref[1]
    for t in range(TRG):
        b = jnp.log(src2_ref[t] + 1.0)
        h0, h1 = minlstm(a, b, 0)
        k0, k1 = minlstm(h0, h1, 18)
        r0 = a + k0
        r1 = b + k1
        d = (r0 - r1) * 0.5
        inv = 1.0 / jnp.sqrt(d * d + 1e-5)
        l0 = d * inv * dp_ref[36] + dp_ref[38]
        l1 = -d * inv * dp_ref[37] + dp_ref[39]
        o = l0 * dp_ref[40] + l1 * dp_ref[41] + dp_ref[42]
        out_ref[pl.ds(t + 1, 1)] = o[None]
        a = o


_f32 = jnp.float32


def _sds(shape):
    return jax.ShapeDtypeStruct(shape, _f32)


def _full(shape):
    return pl.BlockSpec(shape, lambda i: tuple(0 for _ in shape))


_prep = pl.pallas_call(
    _prep_body,
    grid=(NB,),
    in_specs=[
        pl.BlockSpec((2, BP, 16), lambda i: (0, i, 0)),
        _full((16, 64)), _full((64,)), _full((16, 64)), _full((64,)),
    ],
    out_specs=[
        pl.BlockSpec((2, BP, 16), lambda i: (0, i, 0)),
        pl.BlockSpec((2, BP, 64), lambda i: (0, i, 0)),
        pl.BlockSpec((2, BP, 64), lambda i: (0, i, 0)),
    ],
    out_shape=[_sds((2, NP, 16)), _sds((2, NP, 64)), _sds((2, NP, 64))],
)

_mid = pl.pallas_call(
    _mid_body,
    grid=(NB,),
    in_specs=[
        pl.BlockSpec((2, BP, 80), lambda i: (0, i, 0)),
        _full((64, 16)), _full((16,)), _full((64, 16)), _full((16,)),
    ],
    out_specs=[
        pl.BlockSpec((2, BP, 16), lambda i: (0, i, 0)),
        pl.BlockSpec((2, BP, 16), lambda i: (0, i, 0)),
    ],
    out_shape=[_sds((2, NP, 16)), _sds((2, NP, 16))],
)

_post = pl.pallas_call(
    _post_body,
    grid=(NB,),
    in_specs=[
        pl.BlockSpec((2, BP, 32), lambda i: (0, i, 0)),
        pl.BlockSpec((2, BP, 16), lambda i: (0, i, 0)),
        _full((16,)), _full((16,)), _full((16,)),
        pl.BlockSpec(memory_space=pltpu.SMEM),
    ],
    out_specs=pl.BlockSpec((2, BP), lambda i: (0, i)),
    out_shape=_sds((2, NP)),
)

_dec = pl.pallas_call(
    _dec_body,
    in_specs=[
        pl.BlockSpec((2, NR, NL), lambda: (0, 0, 0)),
        pl.BlockSpec((TRG, NR, NL), lambda: (0, 0, 0)),
        pl.BlockSpec(memory_space=pltpu.SMEM),
    ],
    out_specs=pl.BlockSpec((TRG + 1, NR, NL), lambda: (0, 0, 0)),
    out_shape=_sds((TRG + 1, NR, NL)),
)


def kernel(src1, src2, edge_index, params):
    p = params
    idt = edge_index.dtype

    xsel = jnp.transpose(src1[:, jnp.array([0, SEQ - 1]), :], (1, 0, 2))
    xsel = jnp.pad(xsel, ((0, 0), (0, NP - N), (0, 0)))
    loop = jnp.arange(N, dtype=idt)
    epad = jnp.full((EPAD - ETOT,), N, idt)
    srcf = jnp.concatenate([edge_index[0], loop, epad])
    dstf = jnp.concatenate([edge_index[1], loop, epad])
    src2t = jnp.pad(jnp.transpose(src2[:, :, 0]), ((0, 0), (0, NP - N)))

    dp = jnp.concatenate([
        p['lstm_Wf'].reshape(-1), p['lstm_bf'],
        p['lstm_Wi'].reshape(-1), p['lstm_bi'],
        p['lstm_Wh'].reshape(-1), p['lstm_bh'],
        p['lstm1_Wf'].reshape(-1), p['lstm1_bf'],
        p['lstm1_Wi'].reshape(-1), p['lstm1_bi'],
        p['lstm1_Wh'].reshape(-1), p['lstm1_bh'],
        p['ln1_g'], p['ln1_b'],
        p['fc_out_W'].reshape(-1), p['fc_out_b'],
        p['enc_fc_b'], jnp.zeros((4,), _f32),
    ])

    x_tbl, fs0, fd0 = _prep(xsel, p['gnn0_Wsrc'], p['gnn0_bsrc'],
                            p['gnn0_Wdst'], p['gnn0_bdst'])
    acc0 = _edge0(srcf, dstf, fs0.reshape(2 * NP, 64),
                  fd0.reshape(2 * NP, 64), p['gnn0_attn'])
    fs1, fd1 = _mid(acc0, p['gnn1_Wsrc'], p['gnn1_bsrc'],
                    p['gnn1_Wdst'], p['gnn1_bdst'])
    acc1 = _edge1(srcf, dstf, fs1.reshape(2 * NP, 16),
                  fd1.reshape(2 * NP, 16), p['gnn1_attn'])
    encs = _post(acc1, x_tbl, p['ln0_g'], p['ln0_b'], p['enc_fc_W'][:, 0], dp)
    out13 = _dec(encs.reshape(2, NR, NL), src2t.reshape(TRG, NR, NL), dp)
    return jnp.transpose(out13.reshape(TRG + 1, NP)[:, :N])[:, :, None]


# speedup vs baseline: 2.2953x; 1.0025x over previous
import functools

import jax
import jax.numpy as jnp
from jax import lax
from jax.experimental import pallas as pl
from jax.experimental.pallas import tpu as pltpu
from jax.experimental.pallas import tpu_sc as plsc

N = 10000
E = 160000
ETOT = E + N
SEQ = 24
TRG = 12
F = 16
H0 = 4
NP = 10240
NC = 2
NT = 16
CH = 128
EPT = 10752
NCH = EPT // CH
EPAD = NT * EPT
RPT = NP // NT

_mesh = plsc.VectorSubcoreMesh(core_axis_name="c", subcore_axis_name="s",
                               num_cores=NC, num_subcores=NT)


def _edge_pass(width, heads):
    W = width
    WA = width + 16

    @functools.partial(
        pl.kernel,
        out_type=jax.ShapeDtypeStruct((NC, NP, WA), jnp.float32),
        mesh=_mesh,
        compiler_params=pltpu.CompilerParams(
            use_tc_tiling_on_sc=False, needs_layout_passes=False),
        scratch_types=[
            pltpu.VMEM((EPT,), jnp.int32),
            pltpu.VMEM((EPT,), jnp.int32),
            pltpu.VMEM((CH,), jnp.int32),
            pltpu.VMEM((CH,), jnp.int32),
            pltpu.VMEM((CH, W), jnp.float32),
            pltpu.VMEM((CH, W), jnp.float32),
            pltpu.VMEM((CH, W), jnp.float32),
            pltpu.VMEM((CH, W), jnp.float32),
            pltpu.VMEM((CH, WA), jnp.float32),
            pltpu.VMEM((64, WA), jnp.float32),
            pltpu.VMEM((heads, 16), jnp.float32),
            pltpu.VMEM_SHARED((NP, WA), jnp.float32),
            pltpu.SemaphoreType.DMA,
            pltpu.SemaphoreType.DMA,
        ],
    )
    def edge_kernel(src_hbm, dst_hbm, fs_hbm, fd_hbm, attn_hbm, out_hbm,
                    srcall, dstall, gd0, gd1, fsr0, fdr0, fsr1, fdr1,
                    rows, zbuf, attn_v, acc, sem0, sem1):
        c = lax.axis_index("c")
        s = lax.axis_index("s")
        pltpu.sync_copy(attn_hbm, attn_v)

        zv = jnp.zeros((16,), jnp.float32)

        def zero_stage(i, carry):
            for j in range(WA // 16):
                zbuf[i, pl.ds(j * 16, 16)] = zv
            return carry

        lax.fori_loop(0, 64, zero_stage, 0)

        def zero_acc(i, carry):
            pltpu.sync_copy(zbuf, acc.at[pl.ds(s * RPT + i * 64, 64)])
            return carry

        lax.fori_loop(0, RPT // 64, zero_acc, 0)

        pltpu.sync_copy(src_hbm.at[pl.ds(s * EPT, EPT)], srcall)
        pltpu.sync_copy(dst_hbm.at[pl.ds(s * EPT, EPT)], dstall)
        off = c * NP

        @plsc.parallel_loop(0, EPT // 16, unroll=8)
        def addoff(i):
            srcall[pl.ds(i * 16, 16)] = srcall[pl.ds(i * 16, 16)] + off

        plsc.subcore_barrier()

        attn_h = [attn_v[h] for h in range(heads)]
        lane = lax.broadcasted_iota(jnp.int32, (16,), 0)
        masks = [(lane == h).astype(jnp.float32) for h in range(heads)]

        def issue(k, gd, fsr, fdr, sem):
            for j in range(CH // 16):
                gd[pl.ds(j * 16, 16)] = dstall[pl.ds(k * CH + j * 16, 16)] + off
            hs = pltpu.async_copy(fs_hbm.at[srcall.at[pl.ds(k * CH, CH)]],
                                  fsr, sem)
            hd = pltpu.async_copy(fd_hbm.at[gd], fdr, sem)
            return hs, hd

        def consume(k, fsr, fdr):
            @plsc.parallel_loop(0, CH, unroll=8)
            def edge(e):
                den = None
                for h in range(heads):
                    a = fsr[e, pl.ds(h * 16, 16)]
                    b = fdr[e, pl.ds(h * 16, 16)]
                    q = a + b
                    ql = jnp.maximum(q, 0.0) + 0.2 * jnp.minimum(q, 0.0)
                    lg = jnp.sum(ql * attn_h[h])
                    ex = jnp.exp(jnp.broadcast_to(lg, (16,)))
                    rows[e, pl.ds(h * 16, 16)] = a * ex
                    dh = ex * masks[h]
                    den = dh if den is None else den + dh
                rows[e, pl.ds(W, 16)] = den
            pltpu.sync_copy(rows, acc.at[dstall.at[pl.ds(k * CH, CH)]],
                            add=True)

        def chunk2(j, carry):
            a = 2 * j
            b = a + 1
            ha_s, ha_d = issue(a, gd0, fsr0, fdr0, sem0)
            hb_s, hb_d = issue(b, gd1, fsr1, fdr1, sem1)
            ha_s.wait()
            ha_d.wait()
            consume(a, fsr0, fdr0)
            hb_s.wait()
            hb_d.wait()
            consume(b, fsr1, fdr1)
            return carry

        lax.fori_loop(0, NCH // 2, chunk2, 0)
        plsc.subcore_barrier()
        pltpu.sync_copy(acc.at[pl.ds(s * RPT, RPT)],
                        out_hbm.at[c, pl.ds(s * RPT, RPT)])

    return edge_kernel


_edge0 = _edge_pass(64, 4)
_edge1 = _edge_pass(16, 1)


BP = 2048
NB = NP // BP


def _prep_body(xsel_ref, w0s_ref, b0s_ref, w0d_ref, b0d_ref,
               x_ref, fs_ref, fd_ref):
    a = xsel_ref[...]
    lane = lax.broadcasted_iota(jnp.int32, a.shape, 2)
    x = jnp.where(lane == 0, a, jnp.log(a + 1.0) * (1.0 / jnp.log(10.0)))
    x_ref[...] = x
    x2 = x.reshape(2 * BP, 16)
    fs_ref[...] = (jnp.dot(x2, w0s_ref[...], preferred_element_type=jnp.float32)
                   + b0s_ref[...]).reshape(2, BP, 64)
    fd_ref[...] = (jnp.dot(x2, w0d_ref[...], preferred_element_type=jnp.float32)
                   + b0d_ref[...]).reshape(2, BP, 64)


def _mid_body(acc_ref, w1s_ref, b1s_ref, w1d_ref, b1d_ref, fs1_ref, fd1_ref):
    r = acc_ref[...]
    hs = []
    for h in range(H0):
        num = r[..., h * 16:(h + 1) * 16]
        den = r[..., 64 + h:65 + h]
        hs.append(num / (den + 1e-9))
    h0 = jnp.concatenate(hs, axis=-1).reshape(2 * BP, 64)
    fs1_ref[...] = (jnp.dot(h0, w1s_ref[...], preferred_element_type=jnp.float32)
                    + b1s_ref[...]).reshape(2, BP, 16)
    fd1_ref[...] = (jnp.dot(h0, w1d_ref[...], preferred_element_type=jnp.float32)
                    + b1d_ref[...]).reshape(2, BP, 16)


def _post_body(acc1_ref, x_ref, ln0g_ref, ln0b_ref, encw_ref, dp_ref,
               enc_ref):
    r = acc1_ref[...]
    h1 = r[..., :16] / (r[..., 16:17] + 1e-9)
    g = x_ref[...] + h1
    mu = jnp.mean(g, axis=-1, keepdims=True)
    var = jnp.mean((g - mu) ** 2, axis=-1, keepdims=True)
    ln = (g - mu) / jnp.sqrt(var + 1e-5) * ln0g_ref[...] + ln0b_ref[...]
    enc_ref[...] = jnp.sum(ln * encw_ref[...], axis=-1) + dp_ref[43]


NR = 8
NL = NP // NR


def _dec_body(enc_ref, src2_ref, dp_ref, out_ref):
    out_ref[pl.ds(0, 1)] = enc_ref[pl.ds(0, 1)]

    def lin2(a, b, i):
        u0 = a * dp_ref[i] + b * dp_ref[i + 2] + dp_ref[i + 4]
        u1 = a * dp_ref[i + 1] + b * dp_ref[i + 3] + dp_ref[i + 5]
        return u0, u1

    def minlstm(a, b, base):
        uf0, uf1 = lin2(a, b, base)
        ui0, ui1 = lin2(a, b, base + 6)
        ug0, ug1 = lin2(a, b, base + 12)
        f0 = jax.nn.sigmoid(uf0)
        f1 = jax.nn.sigmoid(uf1)
        i0 = jax.nn.sigmoid(ui0)
        i1 = jax.nn.sigmoid(ui1)
        return i0 * ug0 / (f0 + i0 + 1e-9), i1 * ug1 / (f1 + i1 + 1e-9)

    a = enc_ref[1]
    for t in range(TRG):
        b = jnp.log(src2_ref[t] + 1.0)
        h0, h1 = minlstm(a, b, 0)
        k0, k1 = minlstm(h0, h1, 18)
        r0 = a + k0
        r1 = b + k1
        d = (r0 - r1) * 0.5
        inv = 1.0 / jnp.sqrt(d * d + 1e-5)
        l0 = d * inv * dp_ref[36] + dp_ref[38]
        l1 = -d * inv * dp_ref[37] + dp_ref[39]
        o = l0 * dp_ref[40] + l1 * dp_ref[41] + dp_ref[42]
        out_ref[pl.ds(t + 1, 1)] = o[None]
        a = o


_f32 = jnp.float32


def _sds(shape):
    return jax.ShapeDtypeStruct(shape, _f32)


def _full(shape):
    return pl.BlockSpec(shape, lambda i: tuple(0 for _ in shape))


_prep = pl.pallas_call(
    _prep_body,
    grid=(NB,),
    in_specs=[
        pl.BlockSpec((2, BP, 16), lambda i: (0, i, 0)),
        _full((16, 64)), _full((64,)), _full((16, 64)), _full((64,)),
    ],
    out_specs=[
        pl.BlockSpec((2, BP, 16), lambda i: (0, i, 0)),
        pl.BlockSpec((2, BP, 64), lambda i: (0, i, 0)),
        pl.BlockSpec((2, BP, 64), lambda i: (0, i, 0)),
    ],
    out_shape=[_sds((2, NP, 16)), _sds((2, NP, 64)), _sds((2, NP, 64))],
)

_mid = pl.pallas_call(
    _mid_body,
    grid=(NB,),
    in_specs=[
        pl.BlockSpec((2, BP, 80), lambda i: (0, i, 0)),
        _full((64, 16)), _full((16,)), _full((64, 16)), _full((16,)),
    ],
    out_specs=[
        pl.BlockSpec((2, BP, 16), lambda i: (0, i, 0)),
        pl.BlockSpec((2, BP, 16), lambda i: (0, i, 0)),
    ],
    out_shape=[_sds((2, NP, 16)), _sds((2, NP, 16))],
)

_post = pl.pallas_call(
    _post_body,
    grid=(NB,),
    in_specs=[
        pl.BlockSpec((2, BP, 32), lambda i: (0, i, 0)),
        pl.BlockSpec((2, BP, 16), lambda i: (0, i, 0)),
        _full((16,)), _full((16,)), _full((16,)),
        pl.BlockSpec(memory_space=pltpu.SMEM),
    ],
    out_specs=pl.BlockSpec((2, BP), lambda i: (0, i)),
    out_shape=_sds((2, NP)),
)

_dec = pl.pallas_call(
    _dec_body,
    in_specs=[
        pl.BlockSpec((2, NR, NL), lambda: (0, 0, 0)),
        pl.BlockSpec((TRG, NR, NL), lambda: (0, 0, 0)),
        pl.BlockSpec(memory_space=pltpu.SMEM),
    ],
    out_specs=pl.BlockSpec((TRG + 1, NR, NL), lambda: (0, 0, 0)),
    out_shape=_sds((TRG + 1, NR, NL)),
)


def kernel(src1, src2, edge_index, params):
    p = params
    idt = edge_index.dtype

    xsel = jnp.transpose(src1[:, jnp.array([0, SEQ - 1]), :], (1, 0, 2))
    xsel = jnp.pad(xsel, ((0, 0), (0, NP - N), (0, 0)))
    loop = jnp.arange(N, dtype=idt)
    epad = jnp.full((EPAD - ETOT,), N, idt)
    srcf = jnp.concatenate([edge_index[0], loop, epad])
    dstf = jnp.concatenate([edge_index[1], loop, epad])
    src2t = jnp.pad(jnp.transpose(src2[:, :, 0]), ((0, 0), (0, NP - N)))

    dp = jnp.concatenate([
        p['lstm_Wf'].reshape(-1), p['lstm_bf'],
        p['lstm_Wi'].reshape(-1), p['lstm_bi'],
        p['lstm_Wh'].reshape(-1), p['lstm_bh'],
        p['lstm1_Wf'].reshape(-1), p['lstm1_bf'],
        p['lstm1_Wi'].reshape(-1), p['lstm1_bi'],
        p['lstm1_Wh'].reshape(-1), p['lstm1_bh'],
        p['ln1_g'], p['ln1_b'],
        p['fc_out_W'].reshape(-1), p['fc_out_b'],
        p['enc_fc_b'], jnp.zeros((4,), _f32),
    ])

    x_tbl, fs0, fd0 = _prep(xsel, p['gnn0_Wsrc'], p['gnn0_bsrc'],
                            p['gnn0_Wdst'], p['gnn0_bdst'])
    acc0 = _edge0(srcf, dstf, fs0.reshape(2 * NP, 64),
                  fd0.reshape(2 * NP, 64), p['gnn0_attn'])
    fs1, fd1 = _mid(acc0, p['gnn1_Wsrc'], p['gnn1_bsrc'],
                    p['gnn1_Wdst'], p['gnn1_bdst'])
    acc1 = _edge1(srcf, dstf, fs1.reshape(2 * NP, 16),
                  fd1.reshape(2 * NP, 16), p['gnn1_attn'])
    encs = _post(acc1, x_tbl, p['ln0_g'], p['ln0_b'], p['enc_fc_W'][:, 0], dp)
    out13 = _dec(encs.reshape(2, NR, NL), src2t.reshape(TRG, NR, NL), dp)
    return jnp.transpose(out13.reshape(TRG + 1, NP)[:, :N])[:, :, None]
